# baseline (speedup 1.0000x reference)
import functools

import jax
import jax.numpy as jnp
from jax.experimental import pallas as pl
from jax.experimental.pallas import tpu as pltpu

_NQ = 8
_K = 1024
_D = 512
_TT = 512

_DN = (((1,), (1,)), ((), ()))


def _rvq_kernel(x_ref, hi_ref, mid_ref, lo_ref, n_ref,
                quant_ref, codes_ref, subq_ref, loss_ref):
    b = pl.program_id(0)
    t = pl.program_id(1)

    @pl.when(jnp.logical_and(b == 0, t == 0))
    def _init():
        loss_ref[...] = jnp.zeros((1, 1), jnp.float32)

    r = x_ref[0]
    qsum = jnp.zeros((_TT, _D), dtype=jnp.float32)
    loss_cols = jnp.zeros((_TT, _NQ), jnp.float32)
    codes_acc = jnp.zeros((_TT, _NQ), jnp.int32)
    iota_k = jax.lax.broadcasted_iota(jnp.int32, (_TT, _K), 1)
    iota_q = jax.lax.broadcasted_iota(jnp.int32, (_TT, _NQ), 1)

    for i in range(_NQ):
        a = jnp.sum(r * r, axis=-1, keepdims=True)
        e = jax.lax.dot_general(r.astype(jnp.bfloat16), hi_ref[i],
                                dimension_numbers=_DN,
                                preferred_element_type=jnp.float32)
        d = a - 2.0 * e + n_ref[i:i + 1, :]
        idx = jnp.argmin(d, axis=-1).astype(jnp.int32)
        dmin = jnp.min(d, axis=-1, keepdims=True)
        codes_acc = jnp.where(iota_q == i, idx[:, None], codes_acc)
        loss_cols = jnp.where(iota_q == i, dmin, loss_cols)
        onehot = (idx[:, None] == iota_k).astype(jnp.bfloat16)
        q = jnp.dot(onehot, hi_ref[i], preferred_element_type=jnp.float32)
        q = q + jnp.dot(onehot, mid_ref[i], preferred_element_type=jnp.float32)
        q = q + jnp.dot(onehot, lo_ref[i], preferred_element_type=jnp.float32)
        subq_ref[i, 0, :, :] = q.T
        qsum = qsum + q
        r = r - q

    quant_ref[0] = qsum
    codes_ref[0] = codes_acc
    loss_ref[...] += jnp.sum(loss_cols, keepdims=True)


@functools.partial(jax.jit, static_argnames=())
def kernel(x, codebooks):
    B, T, D = x.shape
    NQ, K, _ = codebooks.shape

    n = jnp.stack([jnp.sum(codebooks[i] * codebooks[i], axis=-1)
                   for i in range(NQ)])

    u = jax.lax.bitcast_convert_type(codebooks, jnp.uint32)
    lsb = (u >> 16) & jnp.uint32(1)
    hi32 = jax.lax.bitcast_convert_type(
        (u + jnp.uint32(0x7FFF) + lsb) & jnp.uint32(0xFFFF0000), jnp.float32)
    rem = codebooks - hi32
    mid32 = jax.lax.bitcast_convert_type(
        jax.lax.bitcast_convert_type(rem, jnp.uint32) & jnp.uint32(0xFFFF0000),
        jnp.float32)
    lo32 = rem - mid32
    cb_hi = hi32.astype(jnp.bfloat16)
    cb_mid = mid32.astype(jnp.bfloat16)
    cb_lo = lo32.astype(jnp.bfloat16)

    grid = (B, T // _TT)
    quant, codes_t, subq, loss = pl.pallas_call(
        _rvq_kernel,
        grid=grid,
        in_specs=[
            pl.BlockSpec((1, _TT, D), lambda b, t: (b, t, 0)),
            pl.BlockSpec((NQ, K, D), lambda b, t: (0, 0, 0)),
            pl.BlockSpec((NQ, K, D), lambda b, t: (0, 0, 0)),
            pl.BlockSpec((NQ, K, D), lambda b, t: (0, 0, 0)),
            pl.BlockSpec((NQ, K), lambda b, t: (0, 0)),
        ],
        out_specs=[
            pl.BlockSpec((1, _TT, D), lambda b, t: (b, t, 0)),
            pl.BlockSpec((1, _TT, NQ), lambda b, t: (b, t, 0)),
            pl.BlockSpec((NQ, 1, D, _TT), lambda b, t: (0, b, 0, t)),
            pl.BlockSpec((1, 1), lambda b, t: (0, 0)),
        ],
        out_shape=[
            jax.ShapeDtypeStruct((B, T, D), jnp.float32),
            jax.ShapeDtypeStruct((B, T, NQ), jnp.int32),
            jax.ShapeDtypeStruct((NQ, B, D, T), jnp.float32),
            jax.ShapeDtypeStruct((1, 1), jnp.float32),
        ],
        compiler_params=pltpu.CompilerParams(
            dimension_semantics=("arbitrary", "arbitrary"),
        ),
    )(x, cb_hi, cb_mid, cb_lo, n)

    codes = jnp.transpose(codes_t, (2, 0, 1))
    commit_loss = (loss[0, 0] / jnp.float32(B * T * D)) / jnp.float32(NQ)
    return quant, codes, commit_loss, subq


# speedup vs baseline: 1.0211x; 1.0211x over previous
import functools

import jax
import jax.numpy as jnp
from jax.experimental import pallas as pl
from jax.experimental.pallas import tpu as pltpu

_NQ = 8
_K = 1024
_D = 512
_TT = 512

_DN = (((1,), (1,)), ((), ()))


def _rvq_kernel(x_ref, hi_ref, mid_ref, lo_ref, n_ref,
                quant_ref, codes_ref, subq_ref, loss_ref):
    b = pl.program_id(0)
    t = pl.program_id(1)

    @pl.when(jnp.logical_and(b == 0, t == 0))
    def _init():
        loss_ref[...] = jnp.zeros((1, 1), jnp.float32)

    r = x_ref[0]
    qsum = jnp.zeros((_TT, _D), dtype=jnp.float32)
    loss_part = jnp.zeros((1, 1), jnp.float32)
    codes_acc = jnp.zeros((_TT, _NQ), jnp.int32)
    iota_k = jax.lax.broadcasted_iota(jnp.int32, (_TT, _K), 1)
    iota_q = jax.lax.broadcasted_iota(jnp.int32, (_TT, _NQ), 1)

    for i in range(_NQ):
        a = jnp.sum(r * r, axis=-1, keepdims=True)
        e = jax.lax.dot_general(r.astype(jnp.bfloat16), hi_ref[i],
                                dimension_numbers=_DN,
                                preferred_element_type=jnp.float32)
        d = a - 2.0 * e + n_ref[i:i + 1, :]
        idx = jnp.argmin(d, axis=-1).astype(jnp.int32)
        codes_acc = jnp.where(iota_q == i, idx[:, None], codes_acc)
        onehot = (idx[:, None] == iota_k).astype(jnp.bfloat16)
        q = jnp.dot(onehot, hi_ref[i], preferred_element_type=jnp.float32)
        q = q + jnp.dot(onehot, mid_ref[i], preferred_element_type=jnp.float32)
        q = q + jnp.dot(onehot, lo_ref[i], preferred_element_type=jnp.float32)
        loss_part = loss_part + jnp.sum((q - r) ** 2, keepdims=True)
        subq_ref[i, 0, :, :] = q.T
        qsum = qsum + q
        r = r - q

    quant_ref[0] = qsum
    codes_ref[0] = codes_acc
    loss_ref[...] += loss_part


@functools.partial(jax.jit, static_argnames=())
def kernel(x, codebooks):
    B, T, D = x.shape
    NQ, K, _ = codebooks.shape

    n = jnp.stack([jnp.sum(codebooks[i] * codebooks[i], axis=-1)
                   for i in range(NQ)])

    u = jax.lax.bitcast_convert_type(codebooks, jnp.uint32)
    lsb = (u >> 16) & jnp.uint32(1)
    hi32 = jax.lax.bitcast_convert_type(
        (u + jnp.uint32(0x7FFF) + lsb) & jnp.uint32(0xFFFF0000), jnp.float32)
    rem = codebooks - hi32
    mid32 = jax.lax.bitcast_convert_type(
        jax.lax.bitcast_convert_type(rem, jnp.uint32) & jnp.uint32(0xFFFF0000),
        jnp.float32)
    lo32 = rem - mid32
    cb_hi = hi32.astype(jnp.bfloat16)
    cb_mid = mid32.astype(jnp.bfloat16)
    cb_lo = lo32.astype(jnp.bfloat16)

    grid = (B, T // _TT)
    quant, codes_t, subq, loss = pl.pallas_call(
        _rvq_kernel,
        grid=grid,
        in_specs=[
            pl.BlockSpec((1, _TT, D), lambda b, t: (b, t, 0)),
            pl.BlockSpec((NQ, K, D), lambda b, t: (0, 0, 0)),
            pl.BlockSpec((NQ, K, D), lambda b, t: (0, 0, 0)),
            pl.BlockSpec((NQ, K, D), lambda b, t: (0, 0, 0)),
            pl.BlockSpec((NQ, K), lambda b, t: (0, 0)),
        ],
        out_specs=[
            pl.BlockSpec((1, _TT, D), lambda b, t: (b, t, 0)),
            pl.BlockSpec((1, _TT, NQ), lambda b, t: (b, t, 0)),
            pl.BlockSpec((NQ, 1, D, _TT), lambda b, t: (0, b, 0, t)),
            pl.BlockSpec((1, 1), lambda b, t: (0, 0)),
        ],
        out_shape=[
            jax.ShapeDtypeStruct((B, T, D), jnp.float32),
            jax.ShapeDtypeStruct((B, T, NQ), jnp.int32),
            jax.ShapeDtypeStruct((NQ, B, D, T), jnp.float32),
            jax.ShapeDtypeStruct((1, 1), jnp.float32),
        ],
        compiler_params=pltpu.CompilerParams(
            dimension_semantics=("arbitrary", "arbitrary"),
        ),
    )(x, cb_hi, cb_mid, cb_lo, n)

    codes = jnp.transpose(codes_t, (2, 0, 1))
    commit_loss = (loss[0, 0] / jnp.float32(B * T * D)) / jnp.float32(NQ)
    return quant, codes, commit_loss, subq
